# baseline (speedup 1.0000x reference)
import functools

import jax
import jax.numpy as jnp
from jax import lax
from jax.experimental import pallas as pl
from jax.experimental.pallas import tpu as pltpu
from jax.experimental.pallas import tpu_sc as plsc

N = 2048
HD = 64
NREL = 257
NW = 32
RPW = N // NW
CH = 4
CW = 256
SPAN = 304

_SC_MESH = plsc.VectorSubcoreMesh(
    core_axis_name="c", subcore_axis_name="s", num_cores=2, num_subcores=16)


@functools.partial(
    pl.kernel,
    out_type=jax.ShapeDtypeStruct((N, 1024, 128), jnp.float32),
    mesh=_SC_MESH,
    scratch_types=[
        pltpu.VMEM((SPAN,), jnp.int32),
        pltpu.VMEM((SPAN, 128), jnp.float32),
        pltpu.VMEM((SPAN, 128), jnp.float32),
        pltpu.SemaphoreType.DMA,
    ],
)
def _sc_body(tp0_hbm, tp1_hbm, out_hbm, idx_ref, buf0_ref, buf1_ref, gsem):
    wid = lax.axis_index("s") * 2 + lax.axis_index("c")
    a = 2176 - RPW * wid
    rmin = lax.shift_right_logical(a - (RPW - 1), 1)
    for c in range(CH):
        b = rmin + CW * c - 1023
        for k in range(SPAN // 16):
            n = lax.iota(jnp.int32, 16) + k * 16
            idx_ref[pl.ds(k * 16, 16)] = jnp.clip(b + n, 0, 129)
        copies = []
        for (o, l) in ((0, 128), (128, 128), (256, 48)):
            copies.append(pltpu.async_copy(
                tp0_hbm.at[idx_ref.at[pl.ds(o, l)]],
                buf0_ref.at[pl.ds(o, l)], gsem))
            copies.append(pltpu.async_copy(
                tp1_hbm.at[idx_ref.at[pl.ds(o, l)]],
                buf1_ref.at[pl.ds(o, l)], gsem))
        for cp in copies:
            cp.wait()
        for t in range(RPW):
            off = lax.shift_right_logical(a - t, 1) - rmin
            src = buf0_ref if t % 2 == 0 else buf1_ref
            pltpu.sync_copy(
                src.at[pl.ds(off, CW)],
                out_hbm.at[wid * RPW + t, pl.ds(CW * c, CW), :])


def kernel(len_q, len_k, embedding_table):
    t = embedding_table
    c00 = jnp.concatenate([t[0], t[0]]).reshape(1, 128)
    c22 = jnp.concatenate([t[NREL - 1], t[NREL - 1]]).reshape(1, 128)
    tp0 = jnp.concatenate([c00, t[0:256].reshape(128, 128), c22])
    tp1 = jnp.concatenate([c00, t[1:257].reshape(128, 128), c22])
    out = _sc_body(tp0, tp1)
    return out.reshape(N, N, HD)


# speedup vs baseline: 1.7787x; 1.7787x over previous
import functools

import jax
import jax.numpy as jnp
from jax import lax
from jax.experimental import pallas as pl
from jax.experimental.pallas import tpu as pltpu
from jax.experimental.pallas import tpu_sc as plsc

N = 2048
HD = 64
NREL = 257
BQ = 16
QLEN = 2304
RPW = QLEN // 16

_SC_MESH = plsc.VectorSubcoreMesh(
    core_axis_name="c", subcore_axis_name="s", num_cores=2, num_subcores=16)


@functools.partial(
    pl.kernel,
    out_type=jax.ShapeDtypeStruct((2, QLEN, 128), jnp.float32),
    mesh=_SC_MESH,
    scratch_types=[
        pltpu.VMEM((RPW,), jnp.int32),
        pltpu.VMEM((RPW, 128), jnp.float32),
        pltpu.SemaphoreType.DMA,
    ],
)
def _gather_q_sc(tp0_hbm, tp1_hbm, q_hbm, idx_ref, buf_ref, gsem):
    sid = lax.axis_index("s")
    ph = lax.axis_index("c")
    base = sid * RPW
    for k in range(RPW // 16):
        n = lax.iota(jnp.int32, 16) + (base + k * 16)
        idx_ref[pl.ds(k * 16, 16)] = jnp.clip(n - 1023, 0, 129)

    @pl.when(ph == 0)
    def _even():
        c0 = pltpu.async_copy(
            tp0_hbm.at[idx_ref.at[pl.ds(0, 128)]],
            buf_ref.at[pl.ds(0, 128)], gsem)
        c1 = pltpu.async_copy(
            tp0_hbm.at[idx_ref.at[pl.ds(128, 16)]],
            buf_ref.at[pl.ds(128, 16)], gsem)
        c0.wait()
        c1.wait()

    @pl.when(ph == 1)
    def _odd():
        c0 = pltpu.async_copy(
            tp1_hbm.at[idx_ref.at[pl.ds(0, 128)]],
            buf_ref.at[pl.ds(0, 128)], gsem)
        c1 = pltpu.async_copy(
            tp1_hbm.at[idx_ref.at[pl.ds(128, 16)]],
            buf_ref.at[pl.ds(128, 16)], gsem)
        c0.wait()
        c1.wait()

    pltpu.sync_copy(buf_ref, q_hbm.at[ph, pl.ds(base, RPW)])


def _expand_body(delta_ref, q_ref, out_ref):
    d = delta_ref[0]
    base = pl.program_id(0) * BQ
    for r in range(BQ):
        s = 2176 + d - (base + r)
        ph = lax.rem(s, 2)
        r0 = lax.div(s - ph, 2)
        out_ref[r, :, :] = q_ref[ph, pl.ds(r0, 1024), :]


def kernel(len_q, len_k, embedding_table):
    delta = (jnp.asarray(len_k, jnp.int32)
             - jnp.asarray(len_q, jnp.int32)).reshape(1)
    t = embedding_table
    c00 = jnp.concatenate([t[0], t[0]]).reshape(1, 128)
    c22 = jnp.concatenate([t[NREL - 1], t[NREL - 1]]).reshape(1, 128)
    tp0 = jnp.concatenate([c00, t[0:256].reshape(128, 128), c22])
    tp1 = jnp.concatenate([c00, t[1:257].reshape(128, 128), c22])
    q = _gather_q_sc(tp0, tp1)
    out = pl.pallas_call(
        _expand_body,
        grid=(N // BQ,),
        in_specs=[
            pl.BlockSpec(memory_space=pltpu.SMEM),
            pl.BlockSpec((2, QLEN, 128), lambda i: (0, 0, 0)),
        ],
        out_specs=pl.BlockSpec((BQ, 1024, 128), lambda i: (i, 0, 0)),
        out_shape=jax.ShapeDtypeStruct((N, 1024, 128), jnp.float32),
    )(delta, q)
    return out.reshape(N, N, HD)
